# baseline (speedup 1.0000x reference)
import jax
import jax.numpy as jnp
from jax.experimental import pallas as pl
from jax.experimental.pallas import tpu as pltpu

_S = 1


def _lobula_kernel(a_hbm, b_hbm, c_hbm, d_hbm, on_hbm, off_hbm,
                   av, bv, cv, dv, onv, offv, in_sem, out_sem):
    H = av.shape[0]
    rows = H // _S

    def in_copy(i, hbm, vmem, s):
        sl = pl.ds(s * rows, rows)
        return pltpu.make_async_copy(hbm.at[sl], vmem.at[sl], in_sem.at[i, s])

    def out_copy(i, vmem, hbm, s):
        sl = pl.ds(s * rows, rows)
        return pltpu.make_async_copy(vmem.at[sl], hbm.at[sl], out_sem.at[i, s])

    for s in range(_S):
        in_copy(0, a_hbm, av, s).start()
        in_copy(1, b_hbm, bv, s).start()
        in_copy(2, c_hbm, cv, s).start()
        in_copy(3, d_hbm, dv, s).start()

    for s in range(_S):
        sl = pl.ds(s * rows, rows)
        in_copy(0, a_hbm, av, s).wait()
        in_copy(1, b_hbm, bv, s).wait()
        onv[sl] = av[sl] * bv[sl]
        out_copy(0, onv, on_hbm, s).start()
        in_copy(2, c_hbm, cv, s).wait()
        in_copy(3, d_hbm, dv, s).wait()
        offv[sl] = cv[sl] * dv[sl]
        out_copy(1, offv, off_hbm, s).start()

    for s in range(_S):
        out_copy(0, onv, on_hbm, s).wait()
        out_copy(1, offv, off_hbm, s).wait()


def _build(H, W, dtype, interpret=False):
    hbm_spec = pl.BlockSpec(memory_space=pltpu.HBM)
    out_sd = jax.ShapeDtypeStruct((H, W), dtype)
    return pl.pallas_call(
        _lobula_kernel,
        in_specs=[hbm_spec] * 4,
        out_specs=(hbm_spec, hbm_spec),
        out_shape=(out_sd, out_sd),
        scratch_shapes=[pltpu.VMEM((H, W), dtype) for _ in range(6)]
        + [pltpu.SemaphoreType.DMA((4, _S)), pltpu.SemaphoreType.DMA((2, _S))],
        interpret=interpret,
    )


def kernel(tm3Signal, tm2Signal, Mi1Para5Signal, tm1Para5Signal, tm1Para3Signal, Mi1Para3Signal):
    H, W = tm3Signal.shape[2], tm3Signal.shape[3]
    shape2d = (H, W)
    on2d, off2d = _build(H, W, tm3Signal.dtype)(
        tm3Signal.reshape(shape2d),
        tm1Para3Signal.reshape(shape2d),
        tm2Signal.reshape(shape2d),
        Mi1Para3Signal.reshape(shape2d),
    )
    return (on2d.reshape(1, 1, H, W), off2d.reshape(1, 1, H, W))


# speedup vs baseline: 1.0401x; 1.0401x over previous
import jax
import jax.numpy as jnp
from jax.experimental import pallas as pl
from jax.experimental.pallas import tpu as pltpu

_S = 2


def _lobula_kernel(a_hbm, b_hbm, c_hbm, d_hbm, on_hbm, off_hbm,
                   av, bv, cv, dv, onv, offv, in_sem, out_sem):
    H = av.shape[0]
    rows = H // _S

    def in_copy(i, hbm, vmem, s):
        sl = pl.ds(s * rows, rows)
        return pltpu.make_async_copy(hbm.at[sl], vmem.at[sl], in_sem.at[i, s])

    def out_copy(i, vmem, hbm, s):
        sl = pl.ds(s * rows, rows)
        return pltpu.make_async_copy(vmem.at[sl], hbm.at[sl], out_sem.at[i, s])

    for s in range(_S):
        in_copy(0, a_hbm, av, s).start()
        in_copy(1, b_hbm, bv, s).start()
        in_copy(2, c_hbm, cv, s).start()
        in_copy(3, d_hbm, dv, s).start()

    for s in range(_S):
        sl = pl.ds(s * rows, rows)
        in_copy(0, a_hbm, av, s).wait()
        in_copy(1, b_hbm, bv, s).wait()
        onv[sl] = av[sl] * bv[sl]
        out_copy(0, onv, on_hbm, s).start()
        in_copy(2, c_hbm, cv, s).wait()
        in_copy(3, d_hbm, dv, s).wait()
        offv[sl] = cv[sl] * dv[sl]
        out_copy(1, offv, off_hbm, s).start()

    for s in range(_S):
        out_copy(0, onv, on_hbm, s).wait()
        out_copy(1, offv, off_hbm, s).wait()


def _build(H, W, dtype, interpret=False):
    hbm_spec = pl.BlockSpec(memory_space=pltpu.HBM)
    out_sd = jax.ShapeDtypeStruct((H, W), dtype)
    return pl.pallas_call(
        _lobula_kernel,
        in_specs=[hbm_spec] * 4,
        out_specs=(hbm_spec, hbm_spec),
        out_shape=(out_sd, out_sd),
        scratch_shapes=[pltpu.VMEM((H, W), dtype) for _ in range(6)]
        + [pltpu.SemaphoreType.DMA((4, _S)), pltpu.SemaphoreType.DMA((2, _S))],
        compiler_params=pltpu.CompilerParams(
            skip_device_barrier=True,
            disable_bounds_checks=True,
        ),
        interpret=interpret,
    )


def kernel(tm3Signal, tm2Signal, Mi1Para5Signal, tm1Para5Signal, tm1Para3Signal, Mi1Para3Signal):
    H, W = tm3Signal.shape[2], tm3Signal.shape[3]
    shape2d = (H, W)
    on2d, off2d = _build(H, W, tm3Signal.dtype)(
        tm3Signal.reshape(shape2d),
        tm1Para3Signal.reshape(shape2d),
        tm2Signal.reshape(shape2d),
        Mi1Para3Signal.reshape(shape2d),
    )
    return (on2d.reshape(1, 1, H, W), off2d.reshape(1, 1, H, W))
